# Initial kernel scaffold; baseline (speedup 1.0000x reference)
import jax
import jax.numpy as jnp
from jax.experimental import pallas as pl


def kernel(x, edge_index, latent_template, templates_features):
    raise NotImplementedError("write your pallas kernel here")


# speedup vs baseline: 8.1996x; 8.1996x over previous
import functools

import jax
import jax.numpy as jnp
from jax import lax
from jax.experimental import pallas as pl
from jax.experimental.pallas import tpu as pltpu
from jax.experimental.pallas import tpu_sc as plsc

N = 10000
E = 320000
D = 128
K = 16
NT = 10
ALPHA = 0.5

NC = 2
NS = 16
NW = NC * NS
CHUNK = 80
ECH = E // CHUNK
TILE_CHUNKS = ECH // NW


def _sc_body(x_hbm, src_hbm, dst_hbm, z2_hbm, z1_hbm,
             agg_out, deg_out,
             sidx_v, didx_v, rows_v, ones_v, agg_sh, deg_sh, sem):
    c = lax.axis_index("c")
    s = lax.axis_index("s")

    @pl.when(s == 0)
    def _():
        pltpu.sync_copy(z2_hbm, agg_sh)
        pltpu.sync_copy(z1_hbm, deg_sh)

    for i in range(CHUNK // 16):
        ones_v[pl.ds(i * 16, 16)] = jnp.ones((16,), jnp.float32)

    plsc.subcore_barrier()

    wid = c * NS + s
    pltpu.sync_copy(src_hbm.at[wid], sidx_v)
    pltpu.sync_copy(dst_hbm.at[wid], didx_v)

    def chunk_step(j, carry):
        pltpu.async_copy(x_hbm.at[sidx_v.at[j]], rows_v, sem).wait()
        pltpu.sync_copy(rows_v, agg_sh.at[didx_v.at[j]], add=True)
        pltpu.sync_copy(ones_v, deg_sh.at[didx_v.at[j]], add=True)
        return carry

    lax.fori_loop(0, TILE_CHUNKS, chunk_step, 0)

    plsc.subcore_barrier()

    @pl.when(s == 0)
    def _():
        pltpu.sync_copy(agg_sh, agg_out.at[c])
        pltpu.sync_copy(deg_sh, deg_out.at[c])


_sc_aggregate = functools.partial(
    pl.kernel,
    out_type=(
        jax.ShapeDtypeStruct((NC, N, D), jnp.float32),
        jax.ShapeDtypeStruct((NC, N), jnp.float32),
    ),
    mesh=plsc.VectorSubcoreMesh(core_axis_name="c", subcore_axis_name="s"),
    scratch_types=(
        pltpu.VMEM((TILE_CHUNKS, CHUNK), jnp.int32),
        pltpu.VMEM((TILE_CHUNKS, CHUNK), jnp.int32),
        pltpu.VMEM((CHUNK, D), jnp.float32),
        pltpu.VMEM((CHUNK,), jnp.float32),
        pltpu.VMEM_SHARED((N, D), jnp.float32),
        pltpu.VMEM_SHARED((N,), jnp.float32),
        pltpu.SemaphoreType.DMA,
    ),
)(_sc_body)


def _tc_body(degb_ref, degf_ref, agg_ref, fsum_ref, cvec_ref, out_ref):
    h = 1.0 / NT
    deg_all = degf_ref[0, :] + degf_ref[1, :]
    degmax = jnp.maximum(jnp.max(deg_all), 1.0)

    dblk = degb_ref[0, :, :] + degb_ref[1, :, :]
    inv = 1.0 / jnp.maximum(dblk, 1.0)
    a = agg_ref[0] + agg_ref[1]
    s2 = jnp.sum(a * a, axis=1, keepdims=True)
    x2 = s2 * inv * inv
    cross = jnp.dot(a, fsum_ref[...],
                    preferred_element_type=jnp.float32) * inv
    wass = x2 + h * cvec_ref[0:1, :] - (2.0 * h) * cross

    dn = dblk / degmax
    gw = ((dn * dn) * float(NT * NT)
          - 2.0 * dn * cvec_ref[1:2, :]
          + cvec_ref[2:3, :]) * (h * h)
    out_ref[...] = ALPHA * wass + (1.0 - ALPHA) * gw


def kernel(x, edge_index, latent_template, templates_features):
    src = edge_index[0].reshape(NW, TILE_CHUNKS, CHUNK)
    dst = edge_index[1].reshape(NW, TILE_CHUNKS, CHUNK)
    z2 = jnp.zeros((N, D), jnp.float32)
    z1 = jnp.zeros((N,), jnp.float32)

    agg2, deg2 = _sc_aggregate(x, src, dst, z2, z1)

    fsum_t = jnp.sum(templates_features, axis=1).T
    f2sum = jnp.sum(templates_features ** 2, axis=(1, 2))
    t_sum = jnp.sum(latent_template, axis=(1, 2))
    tmpl = 0.5 * (latent_template
                  + jnp.transpose(latent_template, (0, 2, 1)))
    t_sq = jnp.sum(tmpl ** 2, axis=(1, 2))
    cvec = jnp.zeros((8, K), jnp.float32)
    cvec = cvec.at[0].set(f2sum).at[1].set(t_sum).at[2].set(t_sq)

    R = 1000
    deg3 = deg2.reshape(NC, N, 1)

    out = pl.pallas_call(
        _tc_body,
        grid=(N // R,),
        in_specs=[
            pl.BlockSpec((NC, R, 1), lambda i: (0, i, 0)),
            pl.BlockSpec((NC, N), lambda i: (0, 0)),
            pl.BlockSpec((NC, R, D), lambda i: (0, i, 0)),
            pl.BlockSpec((D, K), lambda i: (0, 0)),
            pl.BlockSpec((8, K), lambda i: (0, 0)),
        ],
        out_specs=pl.BlockSpec((R, K), lambda i: (i, 0)),
        out_shape=jax.ShapeDtypeStruct((N, K), jnp.float32),
    )(deg3, deg2, agg2, fsum_t, cvec)
    return out


# speedup vs baseline: 12.3801x; 1.5098x over previous
import functools

import jax
import jax.numpy as jnp
from jax import lax
from jax.experimental import pallas as pl
from jax.experimental.pallas import tpu as pltpu
from jax.experimental.pallas import tpu_sc as plsc

N = 10000
E = 320000
D = 128
K = 16
NT = 10
ALPHA = 0.5

NC = 2
NS = 16
NW = NC * NS
CHUNK = 80
ECH = E // CHUNK
TILE_CHUNKS = ECH // NW
TILE_EDGES = E // NW


def _sc_body(x_hbm, src_hbm, dst_hbm, z2_hbm, z1_hbm,
             agg_out, deg_out,
             sidx_v, didx_v, rows_v, ones_v, agg_sh, deg_sh, sem_a):
    c = lax.axis_index("c")
    s = lax.axis_index("s")

    @pl.when(s == 0)
    def _():
        pltpu.sync_copy(z2_hbm, agg_sh)
        pltpu.sync_copy(z1_hbm, deg_sh)

    for i in range(CHUNK // 16):
        ones_v[pl.ds(i * 16, 16)] = jnp.ones((16,), jnp.float32)

    plsc.subcore_barrier()

    wid = c * NS + s
    pltpu.sync_copy(src_hbm.at[pl.ds(wid * TILE_EDGES, TILE_EDGES)], sidx_v)
    pltpu.sync_copy(dst_hbm.at[wid], didx_v)

    def sidx_at(j):
        return sidx_v.at[pl.ds(j * CHUNK, CHUNK)]

    pltpu.async_copy(x_hbm.at[sidx_at(0)], rows_v.at[0], sem_a)

    def step(j, carry):
        @pl.when(j + 1 < TILE_CHUNKS)
        def _():
            pltpu.async_copy(x_hbm.at[sidx_at(j + 1)],
                             rows_v.at[(j + 1) % 2], sem_a)
        pltpu.make_async_copy(x_hbm.at[sidx_at(0)], rows_v.at[0],
                              sem_a).wait()
        buf = rows_v.at[j % 2]
        pltpu.sync_copy(buf, agg_sh.at[didx_v.at[j]], add=True)
        pltpu.sync_copy(ones_v, deg_sh.at[didx_v.at[j]], add=True)
        return carry

    lax.fori_loop(0, TILE_CHUNKS, step, 0)

    plsc.subcore_barrier()

    @pl.when(s == 0)
    def _():
        pltpu.sync_copy(agg_sh, agg_out.at[c])
        pltpu.sync_copy(deg_sh, deg_out.at[c])


_sc_aggregate = functools.partial(
    pl.kernel,
    out_type=(
        jax.ShapeDtypeStruct((NC, N, D), jnp.float32),
        jax.ShapeDtypeStruct((NC, N), jnp.float32),
    ),
    mesh=plsc.VectorSubcoreMesh(core_axis_name="c", subcore_axis_name="s"),
    scratch_types=(
        pltpu.VMEM((TILE_EDGES,), jnp.int32),
        pltpu.VMEM((TILE_CHUNKS, CHUNK), jnp.int32),
        pltpu.VMEM((2, CHUNK, D), jnp.float32),
        pltpu.VMEM((CHUNK,), jnp.float32),
        pltpu.VMEM_SHARED((N, D), jnp.float32),
        pltpu.VMEM_SHARED((N,), jnp.float32),
        pltpu.SemaphoreType.DMA,
    ),
)(_sc_body)


def _tc_body(degb_ref, degf_ref, agg_ref, fsum_ref, cvec_ref, out_ref):
    h = 1.0 / NT
    deg_all = degf_ref[0, :] + degf_ref[1, :]
    degmax = jnp.maximum(jnp.max(deg_all), 1.0)

    dblk = degb_ref[0, :, :] + degb_ref[1, :, :]
    inv = 1.0 / jnp.maximum(dblk, 1.0)
    a = agg_ref[0] + agg_ref[1]
    s2 = jnp.sum(a * a, axis=1, keepdims=True)
    x2 = s2 * inv * inv
    cross = jnp.dot(a, fsum_ref[...],
                    preferred_element_type=jnp.float32) * inv
    wass = x2 + h * cvec_ref[0:1, :] - (2.0 * h) * cross

    dn = dblk / degmax
    gw = ((dn * dn) * float(NT * NT)
          - 2.0 * dn * cvec_ref[1:2, :]
          + cvec_ref[2:3, :]) * (h * h)
    out_ref[...] = ALPHA * wass + (1.0 - ALPHA) * gw


def kernel(x, edge_index, latent_template, templates_features):
    src = edge_index[0]
    dst = edge_index[1].reshape(NW, TILE_CHUNKS, CHUNK)
    z2 = jnp.zeros((N, D), jnp.float32)
    z1 = jnp.zeros((N,), jnp.float32)

    agg2, deg2 = _sc_aggregate(x, src, dst, z2, z1)

    fsum_t = jnp.sum(templates_features, axis=1).T
    f2sum = jnp.sum(templates_features ** 2, axis=(1, 2))
    t_sum = jnp.sum(latent_template, axis=(1, 2))
    tmpl = 0.5 * (latent_template
                  + jnp.transpose(latent_template, (0, 2, 1)))
    t_sq = jnp.sum(tmpl ** 2, axis=(1, 2))
    cvec = jnp.zeros((8, K), jnp.float32)
    cvec = cvec.at[0].set(f2sum).at[1].set(t_sum).at[2].set(t_sq)

    R = 1000
    deg3 = deg2.reshape(NC, N, 1)

    out = pl.pallas_call(
        _tc_body,
        grid=(N // R,),
        in_specs=[
            pl.BlockSpec((NC, R, 1), lambda i: (0, i, 0)),
            pl.BlockSpec((NC, N), lambda i: (0, 0)),
            pl.BlockSpec((NC, R, D), lambda i: (0, i, 0)),
            pl.BlockSpec((D, K), lambda i: (0, 0)),
            pl.BlockSpec((8, K), lambda i: (0, 0)),
        ],
        out_specs=pl.BlockSpec((R, K), lambda i: (i, 0)),
        out_shape=jax.ShapeDtypeStruct((N, K), jnp.float32),
    )(deg3, deg2, agg2, fsum_t, cvec)
    return out


# speedup vs baseline: 12.5774x; 1.0159x over previous
import functools

import jax
import jax.numpy as jnp
from jax import lax
from jax.experimental import pallas as pl
from jax.experimental.pallas import tpu as pltpu
from jax.experimental.pallas import tpu_sc as plsc

N = 10000
E = 320000
D = 128
K = 16
NT = 10
ALPHA = 0.5

NC = 2
NS = 16
NW = NC * NS
CHUNK = 80
ECH = E // CHUNK
TILE_CHUNKS = ECH // NW
TILE_EDGES = E // NW


NROWS_PER_TILE = N // NS


def _sc_body(x_hbm, src_hbm, dst_hbm, z1_hbm,
             agg_out, deg_out,
             sidx_v, didx_v, rows_v, ones_v, agg_sh, deg_sh, sem_a):
    c = lax.axis_index("c")
    s = lax.axis_index("s")

    def zstep(i, carry):
        rows_v[0, i // 8, pl.ds((i % 8) * 16, 16)] = jnp.zeros(
            (16,), jnp.float32)
        return carry

    lax.fori_loop(0, CHUNK * 8, zstep, 0)
    zbase = s * NROWS_PER_TILE
    for r in range(NROWS_PER_TILE // CHUNK):
        pltpu.sync_copy(rows_v.at[0],
                        agg_sh.at[pl.ds(zbase + r * CHUNK, CHUNK)])
    rem = NROWS_PER_TILE % CHUNK
    if rem:
        pltpu.sync_copy(
            rows_v.at[0].at[pl.ds(0, rem)],
            agg_sh.at[pl.ds(zbase + NROWS_PER_TILE - rem, rem)])

    @pl.when(s == 0)
    def _():
        pltpu.sync_copy(z1_hbm, deg_sh)

    for i in range(CHUNK // 16):
        ones_v[pl.ds(i * 16, 16)] = jnp.ones((16,), jnp.float32)

    plsc.subcore_barrier()

    wid = c * NS + s
    pltpu.sync_copy(src_hbm.at[pl.ds(wid * TILE_EDGES, TILE_EDGES)], sidx_v)
    pltpu.sync_copy(dst_hbm.at[wid], didx_v)

    def sidx_at(j):
        return sidx_v.at[pl.ds(j * CHUNK, CHUNK)]

    pltpu.async_copy(x_hbm.at[sidx_at(0)], rows_v.at[0], sem_a)

    def step(j, carry):
        @pl.when(j + 1 < TILE_CHUNKS)
        def _():
            pltpu.async_copy(x_hbm.at[sidx_at(j + 1)],
                             rows_v.at[(j + 1) % 2], sem_a)
        pltpu.make_async_copy(x_hbm.at[sidx_at(0)], rows_v.at[0],
                              sem_a).wait()
        buf = rows_v.at[j % 2]
        pltpu.sync_copy(buf, agg_sh.at[didx_v.at[j]], add=True)
        pltpu.sync_copy(ones_v, deg_sh.at[didx_v.at[j]], add=True)
        return carry

    lax.fori_loop(0, TILE_CHUNKS, step, 0)

    plsc.subcore_barrier()

    @pl.when(s == 0)
    def _():
        pltpu.sync_copy(agg_sh, agg_out.at[c])
        pltpu.sync_copy(deg_sh, deg_out.at[c])


_sc_aggregate = functools.partial(
    pl.kernel,
    out_type=(
        jax.ShapeDtypeStruct((NC, N, D), jnp.float32),
        jax.ShapeDtypeStruct((NC, N), jnp.float32),
    ),
    mesh=plsc.VectorSubcoreMesh(core_axis_name="c", subcore_axis_name="s"),
    scratch_types=(
        pltpu.VMEM((TILE_EDGES,), jnp.int32),
        pltpu.VMEM((TILE_CHUNKS, CHUNK), jnp.int32),
        pltpu.VMEM((2, CHUNK, D), jnp.float32),
        pltpu.VMEM((CHUNK,), jnp.float32),
        pltpu.VMEM_SHARED((N, D), jnp.float32),
        pltpu.VMEM_SHARED((N,), jnp.float32),
        pltpu.SemaphoreType.DMA,
    ),
)(_sc_body)


def _tc_body(degb_ref, degf_ref, agg_ref, fsum_ref, cvec_ref, out_ref):
    h = 1.0 / NT
    deg_all = degf_ref[0, :] + degf_ref[1, :]
    degmax = jnp.maximum(jnp.max(deg_all), 1.0)

    dblk = degb_ref[0, :, :] + degb_ref[1, :, :]
    inv = 1.0 / jnp.maximum(dblk, 1.0)
    a = agg_ref[0] + agg_ref[1]
    s2 = jnp.sum(a * a, axis=1, keepdims=True)
    x2 = s2 * inv * inv
    cross = jnp.dot(a, fsum_ref[...],
                    preferred_element_type=jnp.float32) * inv
    wass = x2 + h * cvec_ref[0:1, :] - (2.0 * h) * cross

    dn = dblk / degmax
    gw = ((dn * dn) * float(NT * NT)
          - 2.0 * dn * cvec_ref[1:2, :]
          + cvec_ref[2:3, :]) * (h * h)
    out_ref[...] = ALPHA * wass + (1.0 - ALPHA) * gw


def kernel(x, edge_index, latent_template, templates_features):
    src = edge_index[0]
    dst = edge_index[1].reshape(NW, TILE_CHUNKS, CHUNK)
    z1 = jnp.zeros((N,), jnp.float32)

    agg2, deg2 = _sc_aggregate(x, src, dst, z1)

    fsum_t = jnp.sum(templates_features, axis=1).T
    f2sum = jnp.sum(templates_features ** 2, axis=(1, 2))
    t_sum = jnp.sum(latent_template, axis=(1, 2))
    tmpl = 0.5 * (latent_template
                  + jnp.transpose(latent_template, (0, 2, 1)))
    t_sq = jnp.sum(tmpl ** 2, axis=(1, 2))
    cvec = jnp.zeros((8, K), jnp.float32)
    cvec = cvec.at[0].set(f2sum).at[1].set(t_sum).at[2].set(t_sq)

    R = 1000
    deg3 = deg2.reshape(NC, N, 1)

    out = pl.pallas_call(
        _tc_body,
        grid=(N // R,),
        in_specs=[
            pl.BlockSpec((NC, R, 1), lambda i: (0, i, 0)),
            pl.BlockSpec((NC, N), lambda i: (0, 0)),
            pl.BlockSpec((NC, R, D), lambda i: (0, i, 0)),
            pl.BlockSpec((D, K), lambda i: (0, 0)),
            pl.BlockSpec((8, K), lambda i: (0, 0)),
        ],
        out_specs=pl.BlockSpec((R, K), lambda i: (i, 0)),
        out_shape=jax.ShapeDtypeStruct((N, K), jnp.float32),
    )(deg3, deg2, agg2, fsum_t, cvec)
    return out


# speedup vs baseline: 12.7138x; 1.0108x over previous
import functools

import jax
import jax.numpy as jnp
from jax import lax
from jax.experimental import pallas as pl
from jax.experimental.pallas import tpu as pltpu
from jax.experimental.pallas import tpu_sc as plsc

N = 10000
E = 320000
D = 128
K = 16
NT = 10
ALPHA = 0.5

NC = 2
NS = 16
NW = NC * NS
CHUNK = 80
ECH = E // CHUNK
TILE_CHUNKS = ECH // NW
TILE_EDGES = E // NW


NROWS_PER_TILE = N // NS


def _sc_body(x_hbm, src_hbm, dst_hbm, z1_hbm,
             agg_out, deg_out,
             sidx_v, didx_v, rows_v, ones_v, agg_sh, deg_sh, sem_a):
    c = lax.axis_index("c")
    s = lax.axis_index("s")

    def zstep(i, carry):
        rows_v[0, i // 8, pl.ds((i % 8) * 16, 16)] = jnp.zeros(
            (16,), jnp.float32)
        return carry

    lax.fori_loop(0, CHUNK * 8, zstep, 0)
    zbase = s * NROWS_PER_TILE
    for r in range(NROWS_PER_TILE // CHUNK):
        pltpu.sync_copy(rows_v.at[0],
                        agg_sh.at[pl.ds(zbase + r * CHUNK, CHUNK)])
    rem = NROWS_PER_TILE % CHUNK
    if rem:
        pltpu.sync_copy(
            rows_v.at[0].at[pl.ds(0, rem)],
            agg_sh.at[pl.ds(zbase + NROWS_PER_TILE - rem, rem)])

    @pl.when(s == 0)
    def _():
        pltpu.sync_copy(z1_hbm, deg_sh)

    for i in range(CHUNK // 16):
        ones_v[pl.ds(i * 16, 16)] = jnp.ones((16,), jnp.float32)

    plsc.subcore_barrier()

    wid = c * NS + s
    pltpu.sync_copy(src_hbm.at[pl.ds(wid * TILE_EDGES, TILE_EDGES)], sidx_v)
    pltpu.sync_copy(dst_hbm.at[wid], didx_v)

    def sidx_at(j):
        return sidx_v.at[pl.ds(j * CHUNK, CHUNK)]

    pltpu.async_copy(x_hbm.at[sidx_at(0)], rows_v.at[0], sem_a)

    def step(j, carry):
        @pl.when(j + 1 < TILE_CHUNKS)
        def _():
            pltpu.async_copy(x_hbm.at[sidx_at(j + 1)],
                             rows_v.at[(j + 1) % 2], sem_a)
        pltpu.make_async_copy(x_hbm.at[sidx_at(0)], rows_v.at[0],
                              sem_a).wait()
        buf = rows_v.at[j % 2]
        pltpu.sync_copy(buf, agg_sh.at[didx_v.at[j]], add=True)
        pltpu.sync_copy(ones_v, deg_sh.at[didx_v.at[j]], add=True)
        return carry

    lax.fori_loop(0, TILE_CHUNKS, step, 0)

    plsc.subcore_barrier()

    @pl.when(s == 0)
    def _():
        pltpu.sync_copy(agg_sh, agg_out.at[c])
        pltpu.sync_copy(deg_sh, deg_out.at[c])


_sc_aggregate = functools.partial(
    pl.kernel,
    out_type=(
        jax.ShapeDtypeStruct((NC, N, D), jnp.float32),
        jax.ShapeDtypeStruct((NC, N), jnp.float32),
    ),
    mesh=plsc.VectorSubcoreMesh(core_axis_name="c", subcore_axis_name="s"),
    scratch_types=(
        pltpu.VMEM((TILE_EDGES,), jnp.int32),
        pltpu.VMEM((TILE_CHUNKS, CHUNK), jnp.int32),
        pltpu.VMEM((2, CHUNK, D), jnp.float32),
        pltpu.VMEM((CHUNK,), jnp.float32),
        pltpu.VMEM_SHARED((N, D), jnp.float32),
        pltpu.VMEM_SHARED((N,), jnp.float32),
        pltpu.SemaphoreType.DMA,
    ),
)(_sc_body)


def _tc_body(degb_ref, degf_ref, agg_ref, fsum_ref, cvec_ref, out_ref):
    h = 1.0 / NT
    deg_all = degf_ref[0, :] + degf_ref[1, :]
    degmax = jnp.maximum(jnp.max(deg_all), 1.0)

    dblk = degb_ref[0, :, :] + degb_ref[1, :, :]
    inv = 1.0 / jnp.maximum(dblk, 1.0)
    a = agg_ref[0] + agg_ref[1]
    s2 = jnp.sum(a * a, axis=1, keepdims=True)
    x2 = s2 * inv * inv
    cross = jnp.dot(a, fsum_ref[...],
                    preferred_element_type=jnp.float32) * inv
    wass = x2 + h * cvec_ref[0:1, :] - (2.0 * h) * cross

    dn = dblk / degmax
    gw = ((dn * dn) * float(NT * NT)
          - 2.0 * dn * cvec_ref[1:2, :]
          + cvec_ref[2:3, :]) * (h * h)
    out_ref[...] = ALPHA * wass + (1.0 - ALPHA) * gw


def kernel(x, edge_index, latent_template, templates_features):
    src = edge_index[0]
    dst = edge_index[1].reshape(NW, TILE_CHUNKS, CHUNK)
    z1 = jnp.zeros((N,), jnp.float32)

    agg2, deg2 = _sc_aggregate(x, src, dst, z1)

    fsum_t = jnp.sum(templates_features, axis=1).T
    f2sum = jnp.sum(templates_features ** 2, axis=(1, 2))
    t_sum = jnp.sum(latent_template, axis=(1, 2))
    tmpl = 0.5 * (latent_template
                  + jnp.transpose(latent_template, (0, 2, 1)))
    t_sq = jnp.sum(tmpl ** 2, axis=(1, 2))
    cvec = jnp.zeros((8, K), jnp.float32)
    cvec = cvec.at[0].set(f2sum).at[1].set(t_sum).at[2].set(t_sq)

    R = 2000
    deg3 = deg2.reshape(NC, N, 1)

    out = pl.pallas_call(
        _tc_body,
        grid=(N // R,),
        in_specs=[
            pl.BlockSpec((NC, R, 1), lambda i: (0, i, 0)),
            pl.BlockSpec((NC, N), lambda i: (0, 0)),
            pl.BlockSpec((NC, R, D), lambda i: (0, i, 0)),
            pl.BlockSpec((D, K), lambda i: (0, 0)),
            pl.BlockSpec((8, K), lambda i: (0, 0)),
        ],
        out_specs=pl.BlockSpec((R, K), lambda i: (i, 0)),
        out_shape=jax.ShapeDtypeStruct((N, K), jnp.float32),
    )(deg3, deg2, agg2, fsum_t, cvec)
    return out
